# baseline (speedup 1.0000x reference)
import functools

import jax
import jax.numpy as jnp
from jax import lax
from jax.experimental import pallas as pl
from jax.experimental.pallas import tpu as pltpu
from jax.experimental.pallas import tpu_sc as plsc

VOCAB = 100000
EMBED = 32
BATCH = 1024
CTX = 20

NC = 2
NS = 16
NW = NC * NS
B_PER_W = BATCH // NW
IDX_PER_W = B_PER_W * CTX
GCHUNK = 128

VBLK = 2048
NBLK = (VOCAB + VBLK - 1) // VBLK


def _gather_mean(x_flat, emb_table):
    mesh = plsc.VectorSubcoreMesh(core_axis_name="c", subcore_axis_name="s")

    @functools.partial(
        pl.kernel,
        mesh=mesh,
        compiler_params=pltpu.CompilerParams(use_tc_tiling_on_sc=False),
        out_type=jax.ShapeDtypeStruct((BATCH, EMBED), jnp.float32),
        scratch_types=[
            pltpu.VMEM((IDX_PER_W,), jnp.int32),
            pltpu.VMEM((IDX_PER_W, EMBED), jnp.float32),
            pltpu.VMEM((B_PER_W, EMBED), jnp.float32),
            pltpu.SemaphoreType.DMA,
        ],
    )
    def k(x_hbm, tbl_hbm, out_hbm, idx_v, rows_v, avg_v, sem):
        wid = lax.axis_index("s") * NC + lax.axis_index("c")
        base = wid * IDX_PER_W
        pltpu.sync_copy(x_hbm.at[pl.ds(base, IDX_PER_W)], idx_v)
        copies = []
        for g in range(IDX_PER_W // GCHUNK):
            copies.append(
                pltpu.async_copy(
                    tbl_hbm.at[idx_v.at[pl.ds(g * GCHUNK, GCHUNK)]],
                    rows_v.at[pl.ds(g * GCHUNK, GCHUNK)],
                    sem,
                )
            )
        for c in copies:
            c.wait()

        def body(i, carry):
            r0 = i * CTX
            acc0 = rows_v[r0, pl.ds(0, 16)]
            acc1 = rows_v[r0, pl.ds(16, 16)]
            for c in range(1, CTX):
                acc0 = acc0 + rows_v[r0 + c, pl.ds(0, 16)]
                acc1 = acc1 + rows_v[r0 + c, pl.ds(16, 16)]
            avg_v[i, pl.ds(0, 16)] = acc0 * (1.0 / CTX)
            avg_v[i, pl.ds(16, 16)] = acc1 * (1.0 / CTX)
            return carry

        lax.fori_loop(0, B_PER_W, body, 0)
        pltpu.sync_copy(avg_v, out_hbm.at[pl.ds(wid * B_PER_W, B_PER_W)])

    return k(x_flat, emb_table)


def _mm_body(avg_ref, wt_ref, b_ref, out_ref):
    out_ref[...] = (
        jnp.dot(avg_ref[...], wt_ref[...], preferred_element_type=jnp.float32)
        + b_ref[...]
    )


def _project(avg, wt, b2):
    return pl.pallas_call(
        _mm_body,
        grid=(NBLK,),
        in_specs=[
            pl.BlockSpec((BATCH, EMBED), lambda i: (0, 0)),
            pl.BlockSpec((EMBED, VBLK), lambda i: (0, i)),
            pl.BlockSpec((1, VBLK), lambda i: (0, i)),
        ],
        out_specs=pl.BlockSpec((BATCH, VBLK), lambda i: (0, i)),
        out_shape=jax.ShapeDtypeStruct((BATCH, VOCAB), jnp.float32),
    )(avg, wt, b2)


def kernel(x, emb_table, W, b):
    x_flat = x.reshape(-1).astype(jnp.int32)
    avg = _gather_mean(x_flat, emb_table)
    wt = W.T
    b2 = b.reshape(1, VOCAB)
    return _project(avg, wt, b2)


# speedup vs baseline: 1.2092x; 1.2092x over previous
import functools

import jax
import jax.numpy as jnp
from jax import lax
from jax.experimental import pallas as pl
from jax.experimental.pallas import tpu as pltpu
from jax.experimental.pallas import tpu_sc as plsc

VOCAB = 100000
EMBED = 32
BATCH = 1024
CTX = 20

NC = 2
NS = 16
NW = NC * NS
B_PER_W = BATCH // NW
IDX_PER_W = B_PER_W * CTX
GCHUNK = 128

VBLK = 2048
NBLK = (VOCAB + VBLK - 1) // VBLK


def _gather_mean(x_flat, emb_table):
    mesh = plsc.VectorSubcoreMesh(core_axis_name="c", subcore_axis_name="s")

    @functools.partial(
        pl.kernel,
        mesh=mesh,
        compiler_params=pltpu.CompilerParams(use_tc_tiling_on_sc=False),
        out_type=jax.ShapeDtypeStruct((BATCH, EMBED), jnp.float32),
        scratch_types=[
            pltpu.VMEM((IDX_PER_W,), jnp.int32),
            pltpu.VMEM((IDX_PER_W, EMBED), jnp.float32),
            pltpu.VMEM((B_PER_W, EMBED), jnp.float32),
            pltpu.SemaphoreType.DMA,
        ],
    )
    def k(x_hbm, tbl_hbm, out_hbm, idx_v, rows_v, avg_v, sem):
        wid = lax.axis_index("s") * NC + lax.axis_index("c")
        base = wid * IDX_PER_W
        pltpu.sync_copy(x_hbm.at[pl.ds(base, IDX_PER_W)], idx_v)
        copies = []
        for g in range(IDX_PER_W // GCHUNK):
            copies.append(
                pltpu.async_copy(
                    tbl_hbm.at[idx_v.at[pl.ds(g * GCHUNK, GCHUNK)]],
                    rows_v.at[pl.ds(g * GCHUNK, GCHUNK)],
                    sem,
                )
            )
        for c in copies:
            c.wait()

        def body(i, carry):
            r0 = i * CTX
            acc0 = rows_v[r0, pl.ds(0, 16)]
            acc1 = rows_v[r0, pl.ds(16, 16)]
            for c in range(1, CTX):
                acc0 = acc0 + rows_v[r0 + c, pl.ds(0, 16)]
                acc1 = acc1 + rows_v[r0 + c, pl.ds(16, 16)]
            avg_v[i, pl.ds(0, 16)] = acc0 * (1.0 / CTX)
            avg_v[i, pl.ds(16, 16)] = acc1 * (1.0 / CTX)
            return carry

        lax.fori_loop(0, B_PER_W, body, 0)
        pltpu.sync_copy(avg_v, out_hbm.at[pl.ds(wid * B_PER_W, B_PER_W)])

    return k(x_flat, emb_table)


def _mm_body(avg_ref, wt_ref, b_ref, out_ref):
    acc = (
        jnp.dot(avg_ref[...], wt_ref[...], preferred_element_type=jnp.float32)
        + b_ref[...]
    )
    out_ref[...] = acc.astype(jnp.bfloat16)


def _project(avg, wt, b2):
    return pl.pallas_call(
        _mm_body,
        grid=(NBLK,),
        in_specs=[
            pl.BlockSpec((BATCH, EMBED), lambda i: (0, 0)),
            pl.BlockSpec((EMBED, VBLK), lambda i: (0, i)),
            pl.BlockSpec((1, VBLK), lambda i: (0, i)),
        ],
        out_specs=pl.BlockSpec((BATCH, VBLK), lambda i: (0, i)),
        out_shape=jax.ShapeDtypeStruct((BATCH, VOCAB), jnp.bfloat16),
    )(avg, wt, b2)


def kernel(x, emb_table, W, b):
    x_flat = x.reshape(-1).astype(jnp.int32)
    avg = _gather_mean(x_flat, emb_table)
    wt = W.T
    b2 = b.reshape(1, VOCAB)
    return _project(avg, wt, b2).astype(jnp.float32)


# speedup vs baseline: 1.2484x; 1.0324x over previous
import functools

import jax
import jax.numpy as jnp
from jax import lax
from jax.experimental import pallas as pl
from jax.experimental.pallas import tpu as pltpu
from jax.experimental.pallas import tpu_sc as plsc

VOCAB = 100000
EMBED = 32
BATCH = 1024
CTX = 20

NC = 2
NS = 16
NW = NC * NS
B_PER_W = BATCH // NW
IDX_PER_W = B_PER_W * CTX
GCHUNK = 128

BB = 128
NBB = BATCH // BB


def _gather_mean(x_flat, emb_table):
    mesh = plsc.VectorSubcoreMesh(core_axis_name="c", subcore_axis_name="s")

    @functools.partial(
        pl.kernel,
        mesh=mesh,
        compiler_params=pltpu.CompilerParams(use_tc_tiling_on_sc=False),
        out_type=jax.ShapeDtypeStruct((BATCH, EMBED), jnp.float32),
        scratch_types=[
            pltpu.VMEM((IDX_PER_W,), jnp.int32),
            pltpu.VMEM((IDX_PER_W, EMBED), jnp.float32),
            pltpu.VMEM((B_PER_W, EMBED), jnp.float32),
            pltpu.SemaphoreType.DMA,
        ],
    )
    def k(x_hbm, tbl_hbm, out_hbm, idx_v, rows_v, avg_v, sem):
        wid = lax.axis_index("s") * NC + lax.axis_index("c")
        base = wid * IDX_PER_W
        pltpu.sync_copy(x_hbm.at[pl.ds(base, IDX_PER_W)], idx_v)
        copies = []
        for g in range(IDX_PER_W // GCHUNK):
            copies.append(
                pltpu.async_copy(
                    tbl_hbm.at[idx_v.at[pl.ds(g * GCHUNK, GCHUNK)]],
                    rows_v.at[pl.ds(g * GCHUNK, GCHUNK)],
                    sem,
                )
            )
        for c in copies:
            c.wait()

        def body(i, carry):
            r0 = i * CTX
            acc0 = rows_v[r0, pl.ds(0, 16)]
            acc1 = rows_v[r0, pl.ds(16, 16)]
            for c in range(1, CTX):
                acc0 = acc0 + rows_v[r0 + c, pl.ds(0, 16)]
                acc1 = acc1 + rows_v[r0 + c, pl.ds(16, 16)]
            avg_v[i, pl.ds(0, 16)] = acc0 * (1.0 / CTX)
            avg_v[i, pl.ds(16, 16)] = acc1 * (1.0 / CTX)
            return carry

        lax.fori_loop(0, B_PER_W, body, 0)
        pltpu.sync_copy(avg_v, out_hbm.at[pl.ds(wid * B_PER_W, B_PER_W)])

    return k(x_flat, emb_table)


def _mm_body(avg_ref, wt_ref, b_ref, out_ref):
    acc = (
        jnp.dot(avg_ref[...], wt_ref[...], preferred_element_type=jnp.float32)
        + b_ref[...]
    )
    out_ref[...] = acc.astype(jnp.bfloat16)


def _project(avg, wt, b2):
    return pl.pallas_call(
        _mm_body,
        grid=(NBB,),
        in_specs=[
            pl.BlockSpec((BB, EMBED), lambda i: (i, 0)),
            pl.BlockSpec((EMBED, VOCAB), lambda i: (0, 0)),
            pl.BlockSpec((1, VOCAB), lambda i: (0, 0)),
        ],
        out_specs=pl.BlockSpec((BB, VOCAB), lambda i: (i, 0)),
        out_shape=jax.ShapeDtypeStruct((BATCH, VOCAB), jnp.bfloat16),
        compiler_params=pltpu.CompilerParams(
            vmem_limit_bytes=110 * 1024 * 1024,
        ),
    )(avg, wt, b2)


def kernel(x, emb_table, W, b):
    x_flat = x.reshape(-1).astype(jnp.int32)
    avg = _gather_mean(x_flat, emb_table)
    wt = W.T
    b2 = b.reshape(1, VOCAB)
    return _project(avg, wt, b2).astype(jnp.float32)
